# baseline (speedup 1.0000x reference)
import functools

import jax
import jax.numpy as jnp
from jax import lax
from jax.experimental import pallas as pl
from jax.experimental.pallas import tpu as pltpu
from jax.experimental.pallas import tpu_sc as plsc

B, N, K = 16384, 1001, 10

NC, NS, L = 2, 16, 16
NW = NC * NS
ROWS_W = B // NW
GROUPS_W = ROWS_W // L
NGROUPS = B // L


def _sc_body(scores_hbm, cand_hbm, cnt_hbm, pos_hbm, buf, cand_v, cnt_v,
             pos_v):
    wid = lax.axis_index("s") * NC + lax.axis_index("c")
    iota16 = lax.iota(jnp.int32, L)
    neg_inf = jnp.full((L,), -jnp.inf, jnp.float32)
    zeros = jnp.zeros((L,), jnp.float32)

    for k in range(K, 16):
        cand_v[k, :] = zeros

    row_base = iota16 * N

    def group_body(g, carry):
        gi = wid * GROUPS_W + g
        base = gi * L
        pltpu.sync_copy(scores_hbm.at[pl.ds(base * N, L * N)], buf)
        pos = plsc.load_gather(buf, [row_base])

        def col_body(j, c):
            x = plsc.load_gather(buf, [row_base + j])
            y = x
            new = []
            for k in range(K):
                ck = c[k]
                new.append(jnp.maximum(ck, y))
                y = jnp.minimum(ck, y)
            new.append(c[K] + (x > pos).astype(jnp.float32))
            return tuple(new)

        init = tuple([neg_inf] * K) + (zeros,)
        res = lax.fori_loop(0, N, col_body, init)
        for k in range(K):
            cand_v[k, :] = res[k]
        cnt_v[...] = res[K]
        pos_v[...] = pos
        pltpu.sync_copy(cand_v, cand_hbm.at[gi])
        pltpu.sync_copy(cnt_v, cnt_hbm.at[gi])
        pltpu.sync_copy(pos_v, pos_hbm.at[gi])
        return carry

    lax.fori_loop(0, GROUPS_W, group_body, 0)


G2 = 128


def _tc_body(cand_ref, cnt_ref, pos_ref, num_ref, den_ref):
    i = pl.program_id(0)
    v = cand_ref[...]
    p = cnt_ref[...]
    pos = pos_ref[...]
    r = lax.broadcasted_iota(jnp.int32, (G2, 16, 16), 1).astype(jnp.float32)
    p3 = p[:, None, :]
    pos3 = pos[:, None, :]
    w = 1.0 / jnp.log2(r + 2.0)
    bpr = -jnp.log(jnp.clip(jax.nn.sigmoid(pos3 - v), 1e-8))
    valid = (r < float(K)) & (r != p3)
    num = jnp.sum(jnp.where(valid, bpr * w, 0.0))
    den = jnp.sum(10.0 - (p < float(K)).astype(jnp.float32))

    @pl.when(i == 0)
    def _():
        num_ref[...] = jnp.zeros((1, 1), jnp.float32)
        den_ref[...] = jnp.zeros((1, 1), jnp.float32)

    num_ref[...] += num.reshape(1, 1)
    den_ref[...] += den.reshape(1, 1)


def kernel(scores):
    mesh = plsc.VectorSubcoreMesh(core_axis_name="c", subcore_axis_name="s")
    sc = pl.kernel(
        _sc_body,
        mesh=mesh,
        compiler_params=pltpu.CompilerParams(needs_layout_passes=False),
        out_type=[
            jax.ShapeDtypeStruct((NGROUPS, 16, L), jnp.float32),
            jax.ShapeDtypeStruct((NGROUPS, L), jnp.float32),
            jax.ShapeDtypeStruct((NGROUPS, L), jnp.float32),
        ],
        scratch_types=[
            pltpu.VMEM((L * N,), jnp.float32),
            pltpu.VMEM((16, L), jnp.float32),
            pltpu.VMEM((L,), jnp.float32),
            pltpu.VMEM((L,), jnp.float32),
        ],
    )
    cand, cnt, posv = sc(scores.reshape(B * N))

    num, den = pl.pallas_call(
        _tc_body,
        grid=(NGROUPS // G2,),
        in_specs=[
            pl.BlockSpec((G2, 16, L), lambda i: (i, 0, 0)),
            pl.BlockSpec((G2, L), lambda i: (i, 0)),
            pl.BlockSpec((G2, L), lambda i: (i, 0)),
        ],
        out_specs=[
            pl.BlockSpec((1, 1), lambda i: (0, 0)),
            pl.BlockSpec((1, 1), lambda i: (0, 0)),
        ],
        out_shape=[
            jax.ShapeDtypeStruct((1, 1), jnp.float32),
            jax.ShapeDtypeStruct((1, 1), jnp.float32),
        ],
    )(cand, cnt, posv)
    return num[0, 0] / jnp.clip(den[0, 0], 1.0)
